# baseline (speedup 1.0000x reference)
import functools

import jax
import jax.numpy as jnp
from jax import lax
from jax.experimental import pallas as pl
from jax.experimental.pallas import tpu as pltpu
from jax.experimental.pallas import tpu_sc as plsc

_D = 768
_B = 4
_S = 4096
_N = _B * _S
_NC = 2
_NS = 16
_NW = _NC * _NS
_SPAN = _S // _NW
_CHUNK = 32
_NJB = _SPAN // _CHUNK
_NCH = _NJB * _B
_LANES = 16
_VECS_PER_ROW = _D // _LANES


def _make_sc_kernel():
    mesh = plsc.VectorSubcoreMesh(core_axis_name="c", subcore_axis_name="s")

    @functools.partial(
        pl.kernel,
        out_type=jax.ShapeDtypeStruct((_B, _S, _D), jnp.float32),
        mesh=mesh,
        scratch_types=[
            pltpu.VMEM((_B, _SPAN), jnp.int32),
            pltpu.VMEM((_CHUNK, _D), jnp.float32),
            pltpu.VMEM((_CHUNK, _D), jnp.float32),
            pltpu.VMEM((_CHUNK, _D), jnp.float32),
            pltpu.VMEM((_CHUNK, _D), jnp.float32),
            pltpu.VMEM((_CHUNK, _D), jnp.float32),
            pltpu.SemaphoreType.DMA,
            pltpu.SemaphoreType.DMA,
            pltpu.SemaphoreType.DMA,
            pltpu.SemaphoreType.DMA,
            pltpu.SemaphoreType.DMA,
            pltpu.SemaphoreType.DMA,
            pltpu.SemaphoreType.DMA,
            pltpu.SemaphoreType.DMA,
        ],
    )
    def body(ids_hbm, word_hbm, pos_hbm, out_hbm, idx_v,
             rows0, rows1, rows2, pos0, pos1,
             sg0, sg1, sg2, st0, st1, st2, sp0, sp1):
        wid = lax.axis_index("s") * _NC + lax.axis_index("c")
        s0 = wid * _SPAN
        rows = (rows0, rows1, rows2)
        sg = (sg0, sg1, sg2)
        st = (st0, st1, st2)
        pos = (pos0, pos1)
        sp = (sp0, sp1)

        pltpu.sync_copy(ids_hbm.at[:, pl.ds(s0, _SPAN)], idx_v)

        def gstart(c):
            i = c % 3
            b, jb = c % _B, c // _B
            pltpu.async_copy(
                word_hbm.at[idx_v.at[b, pl.ds(jb * _CHUNK, _CHUNK)]],
                rows[i], sg[i])

        def gwait(c):
            i = c % 3
            pltpu.make_async_copy(
                word_hbm.at[pl.ds(0, _CHUNK)], rows[i], sg[i]).wait()

        def stwait(c):
            i = c % 3
            pltpu.make_async_copy(
                rows[i], out_hbm.at[0, pl.ds(0, _CHUNK)], st[i]).wait()

        def pstart(jb):
            pltpu.async_copy(pos_hbm.at[pl.ds(s0 + jb * _CHUNK, _CHUNK)],
                             pos[jb % 2], sp[jb % 2])

        def pwait(jb):
            pltpu.make_async_copy(pos_hbm.at[pl.ds(0, _CHUNK)],
                                  pos[jb % 2], sp[jb % 2]).wait()

        pstart(0)
        gstart(0)
        gstart(1)

        for c in range(_NCH):
            i = c % 3
            b, jb = c % _B, c // _B
            if b == 0:
                if jb + 1 < _NJB:
                    pstart(jb + 1)
                pwait(jb)
            gwait(c)
            if c + 2 < _NCH:
                if c >= 1:
                    stwait(c - 1)
                gstart(c + 2)
            rbuf, pbuf = rows[i], pos[jb % 2]

            def row_step(r, carry, rbuf=rbuf, pbuf=pbuf):
                for k in range(_VECS_PER_ROW):
                    plsc.addupdate(
                        rbuf.at[r, pl.ds(k * _LANES, _LANES)],
                        pbuf[r, pl.ds(k * _LANES, _LANES)],
                    )
                return carry

            lax.fori_loop(0, _CHUNK, row_step, 0, unroll=4)
            pltpu.async_copy(
                rbuf, out_hbm.at[b, pl.ds(s0 + jb * _CHUNK, _CHUNK)],
                st[i])

        for c in (_NCH - 3, _NCH - 2, _NCH - 1):
            stwait(c)

    return body


_sc_kernel = _make_sc_kernel()


@jax.jit
def kernel(input_ids, word_embeddings, position_embeddings):
    return _sc_kernel(input_ids.astype(jnp.int32), word_embeddings,
                      position_embeddings)


# speedup vs baseline: 1.0431x; 1.0431x over previous
import functools

import jax
import jax.numpy as jnp
from jax import lax
from jax.experimental import pallas as pl
from jax.experimental.pallas import tpu as pltpu
from jax.experimental.pallas import tpu_sc as plsc

_D = 768
_B = 4
_S = 4096
_N = _B * _S
_NC = 2
_NS = 16
_NW = _NC * _NS
_SPAN = _S // _NW
_CHUNK = 32
_NJB = _SPAN // _CHUNK
_NCH = _NJB * _B
_LANES = 16
_VECS_PER_ROW = _D // _LANES


def _make_sc_kernel():
    mesh = plsc.VectorSubcoreMesh(core_axis_name="c", subcore_axis_name="s")

    @functools.partial(
        pl.kernel,
        out_type=jax.ShapeDtypeStruct((_B, _S, _D), jnp.float32),
        mesh=mesh,
        scratch_types=[
            pltpu.VMEM((_B, _SPAN), jnp.int32),
            pltpu.VMEM((_CHUNK, _D), jnp.float32),
            pltpu.VMEM((_CHUNK, _D), jnp.float32),
            pltpu.VMEM((_CHUNK, _D), jnp.float32),
            pltpu.VMEM((_CHUNK, _D), jnp.float32),
            pltpu.VMEM((_CHUNK, _D), jnp.float32),
            pltpu.SemaphoreType.DMA,
            pltpu.SemaphoreType.DMA,
            pltpu.SemaphoreType.DMA,
            pltpu.SemaphoreType.DMA,
            pltpu.SemaphoreType.DMA,
            pltpu.SemaphoreType.DMA,
            pltpu.SemaphoreType.DMA,
            pltpu.SemaphoreType.DMA,
        ],
    )
    def body(ids_hbm, word_hbm, pos_hbm, out_hbm, idx_v,
             rows0, rows1, rows2, pos0, pos1,
             sg0, sg1, sg2, st0, st1, st2, sp0, sp1):
        wid = lax.axis_index("s") * _NC + lax.axis_index("c")
        s0 = wid * _SPAN
        rows = (rows0, rows1, rows2)
        sg = (sg0, sg1, sg2)
        st = (st0, st1, st2)
        pos = (pos0, pos1)
        sp = (sp0, sp1)

        pltpu.sync_copy(ids_hbm.at[:, pl.ds(s0, _SPAN)], idx_v)

        def gstart(c):
            i = c % 3
            b, jb = c % _B, c // _B
            pltpu.async_copy(
                word_hbm.at[idx_v.at[b, pl.ds(jb * _CHUNK, _CHUNK)]],
                rows[i], sg[i])

        def gwait(c):
            i = c % 3
            pltpu.make_async_copy(
                word_hbm.at[pl.ds(0, _CHUNK)], rows[i], sg[i]).wait()

        def stwait(c):
            i = c % 3
            pltpu.make_async_copy(
                rows[i], out_hbm.at[0, pl.ds(0, _CHUNK)], st[i]).wait()

        def pstart(jb):
            pltpu.async_copy(pos_hbm.at[pl.ds(s0 + jb * _CHUNK, _CHUNK)],
                             pos[jb % 2], sp[jb % 2])

        def pwait(jb):
            pltpu.make_async_copy(pos_hbm.at[pl.ds(0, _CHUNK)],
                                  pos[jb % 2], sp[jb % 2]).wait()

        pstart(0)
        gstart(0)
        gstart(1)

        for c in range(_NCH):
            i = c % 3
            b, jb = c % _B, c // _B
            if b == 0:
                if jb + 1 < _NJB:
                    pstart(jb + 1)
                pwait(jb)
            gwait(c)
            rbuf, pbuf = rows[i], pos[jb % 2]

            def row_step(r, carry, rbuf=rbuf, pbuf=pbuf):
                for k in range(_VECS_PER_ROW):
                    plsc.addupdate(
                        rbuf.at[r, pl.ds(k * _LANES, _LANES)],
                        pbuf[r, pl.ds(k * _LANES, _LANES)],
                    )
                return carry

            lax.fori_loop(0, _CHUNK, row_step, 0, unroll=4)
            pltpu.async_copy(
                rbuf, out_hbm.at[b, pl.ds(s0 + jb * _CHUNK, _CHUNK)],
                st[i])
            if c + 2 < _NCH:
                if c >= 1:
                    stwait(c - 1)
                gstart(c + 2)

        for c in (_NCH - 3, _NCH - 2, _NCH - 1):
            stwait(c)

    return body


_sc_kernel = _make_sc_kernel()


@jax.jit
def kernel(input_ids, word_embeddings, position_embeddings):
    return _sc_kernel(input_ids.astype(jnp.int32), word_embeddings,
                      position_embeddings)
